# Initial kernel scaffold; baseline (speedup 1.0000x reference)
import jax
import jax.numpy as jnp
from jax.experimental import pallas as pl


def kernel(x, W):
    raise NotImplementedError("write your pallas kernel here")


# speedup vs baseline: 3.5458x; 3.5458x over previous
import functools

import jax
import jax.numpy as jnp
from jax import lax
from jax.experimental import pallas as pl
from jax.experimental.pallas import tpu as pltpu
from jax.experimental.pallas import tpu_sc as plsc

C_IN = 100000
D_MODEL = 64

NC = 2
NS = 16
NW = NC * NS

CHUNK = 128


def _make_gather(n_total: int):
    per_w = n_total // NW
    n_chunks = per_w // CHUNK
    mesh = plsc.VectorSubcoreMesh(core_axis_name="c", subcore_axis_name="s")

    @functools.partial(
        pl.kernel,
        out_type=jax.ShapeDtypeStruct((n_total, D_MODEL), jnp.float32),
        mesh=mesh,
        scratch_types=[
            pltpu.VMEM((per_w,), jnp.int32),
            pltpu.VMEM((CHUNK, D_MODEL), jnp.float32),
            pltpu.SemaphoreType.DMA,
        ],
        compiler_params=pltpu.CompilerParams(use_tc_tiling_on_sc=False),
    )
    def gather_kernel(w_hbm, x_hbm, out_hbm, idx_v, rows_v, sem):
        wid = lax.axis_index("s") * NC + lax.axis_index("c")
        base = wid * per_w
        pltpu.sync_copy(x_hbm.at[pl.ds(base, per_w)], idx_v)

        def chunk_body(j, carry):
            cbase = j * CHUNK
            pltpu.async_copy(
                w_hbm.at[idx_v.at[pl.ds(cbase, CHUNK)]], rows_v, sem
            ).wait()
            pltpu.sync_copy(rows_v, out_hbm.at[pl.ds(base + cbase, CHUNK)])
            return carry

        lax.fori_loop(0, n_chunks, chunk_body, 0)

    return gather_kernel


def kernel(x, W):
    b, s = x.shape
    n_total = b * s
    xf = x.reshape(n_total).astype(jnp.int32)
    out = _make_gather(n_total)(W, xf)
    return out.reshape(b, s, D_MODEL)


# speedup vs baseline: 4.2542x; 1.1998x over previous
import functools

import jax
import jax.numpy as jnp
from jax import lax
from jax.experimental import pallas as pl
from jax.experimental.pallas import tpu as pltpu
from jax.experimental.pallas import tpu_sc as plsc

C_IN = 100000
D_MODEL = 64

NC = 2
NS = 16
NW = NC * NS

CHUNK = 128
K = 5
GROUP = K * CHUNK


def _make_gather(n_total: int):
    per_w = n_total // NW
    n_groups = per_w // GROUP
    assert per_w % GROUP == 0 and n_groups % 2 == 0
    mesh = plsc.VectorSubcoreMesh(core_axis_name="c", subcore_axis_name="s")

    @functools.partial(
        pl.kernel,
        out_type=jax.ShapeDtypeStruct((n_total, D_MODEL), jnp.float32),
        mesh=mesh,
        scratch_types=[
            pltpu.VMEM((per_w,), jnp.int32),
            pltpu.VMEM((GROUP, D_MODEL), jnp.float32),
            pltpu.VMEM((GROUP, D_MODEL), jnp.float32),
            pltpu.SemaphoreType.DMA,
            pltpu.SemaphoreType.DMA,
            pltpu.SemaphoreType.DMA,
            pltpu.SemaphoreType.DMA,
        ],
        compiler_params=pltpu.CompilerParams(use_tc_tiling_on_sc=False),
    )
    def gather_kernel(w_hbm, x_hbm, out_hbm, idx_v, rows_a, rows_b,
                      gsem_a, gsem_b, wsem_a, wsem_b):
        wid = lax.axis_index("s") * NC + lax.axis_index("c")
        base = wid * per_w
        pltpu.sync_copy(x_hbm.at[pl.ds(base, per_w)], idx_v)

        def start_gathers(g, rows, gsem):
            for b in range(K):
                off = g * GROUP + b * CHUNK
                pltpu.async_copy(
                    w_hbm.at[idx_v.at[pl.ds(off, CHUNK)]],
                    rows.at[pl.ds(b * CHUNK, CHUNK)],
                    gsem,
                )

        def drain_gathers(rows, gsem):
            for b in range(K):
                pltpu.make_async_copy(
                    w_hbm.at[idx_v.at[pl.ds(b * CHUNK, CHUNK)]],
                    rows.at[pl.ds(b * CHUNK, CHUNK)],
                    gsem,
                ).wait()

        def drain_write(rows, wsem):
            pltpu.make_async_copy(rows, out_hbm.at[pl.ds(0, GROUP)], wsem).wait()

        def do_group(g, rows, gsem, wsem, nxt_rows, nxt_gsem, nxt_wsem):
            drain_gathers(rows, gsem)

            @pl.when(g >= 1)
            def _():
                drain_write(nxt_rows, nxt_wsem)

            @pl.when(g + 1 < n_groups)
            def _():
                start_gathers(g + 1, nxt_rows, nxt_gsem)

            pltpu.async_copy(rows, out_hbm.at[pl.ds(base + g * GROUP, GROUP)],
                             wsem)

        start_gathers(0, rows_a, gsem_a)

        def pair_body(t, carry):
            do_group(2 * t, rows_a, gsem_a, wsem_a, rows_b, gsem_b, wsem_b)
            do_group(2 * t + 1, rows_b, gsem_b, wsem_b, rows_a, gsem_a, wsem_a)
            return carry

        lax.fori_loop(0, n_groups // 2, pair_body, 0)
        drain_write(rows_b, wsem_b)

    return gather_kernel


def kernel(x, W):
    b, s = x.shape
    n_total = b * s
    xf = x.reshape(n_total).astype(jnp.int32)
    out = _make_gather(n_total)(W, xf)
    return out.reshape(b, s, D_MODEL)


# speedup vs baseline: 4.2719x; 1.0042x over previous
import functools

import jax
import jax.numpy as jnp
from jax import lax
from jax.experimental import pallas as pl
from jax.experimental.pallas import tpu as pltpu
from jax.experimental.pallas import tpu_sc as plsc

C_IN = 100000
D_MODEL = 64

NC = 2
NS = 16
NW = NC * NS

GROUP = 640


def _make_gather(n_total: int):
    per_w = n_total // NW
    n_groups = per_w // GROUP
    assert per_w % GROUP == 0 and n_groups % 2 == 0
    mesh = plsc.VectorSubcoreMesh(core_axis_name="c", subcore_axis_name="s")

    @functools.partial(
        pl.kernel,
        out_type=jax.ShapeDtypeStruct((n_total, D_MODEL), jnp.float32),
        mesh=mesh,
        scratch_types=[
            pltpu.VMEM((per_w,), jnp.int32),
            pltpu.VMEM((GROUP, D_MODEL), jnp.float32),
            pltpu.VMEM((GROUP, D_MODEL), jnp.float32),
            pltpu.SemaphoreType.DMA,
            pltpu.SemaphoreType.DMA,
            pltpu.SemaphoreType.DMA,
            pltpu.SemaphoreType.DMA,
        ],
        compiler_params=pltpu.CompilerParams(use_tc_tiling_on_sc=False),
    )
    def gather_kernel(w_hbm, x_hbm, out_hbm, idx_v, rows_a, rows_b,
                      gsem_a, gsem_b, wsem_a, wsem_b):
        wid = lax.axis_index("s") * NC + lax.axis_index("c")
        base = wid * per_w
        pltpu.sync_copy(x_hbm.at[pl.ds(base, per_w)], idx_v)

        def start_gather(g, rows, gsem):
            pltpu.async_copy(
                w_hbm.at[idx_v.at[pl.ds(g * GROUP, GROUP)]], rows, gsem)

        def drain_gather(rows, gsem):
            pltpu.make_async_copy(
                w_hbm.at[idx_v.at[pl.ds(0, GROUP)]], rows, gsem).wait()

        def drain_write(rows, wsem):
            pltpu.make_async_copy(rows, out_hbm.at[pl.ds(0, GROUP)], wsem).wait()

        def do_group(g, rows, gsem, wsem, nxt_rows, nxt_gsem, nxt_wsem):
            drain_gather(rows, gsem)

            @pl.when(g >= 1)
            def _():
                drain_write(nxt_rows, nxt_wsem)

            @pl.when(g + 1 < n_groups)
            def _():
                start_gather(g + 1, nxt_rows, nxt_gsem)

            pltpu.async_copy(rows, out_hbm.at[pl.ds(base + g * GROUP, GROUP)],
                             wsem)

        start_gather(0, rows_a, gsem_a)

        def pair_body(t, carry):
            do_group(2 * t, rows_a, gsem_a, wsem_a, rows_b, gsem_b, wsem_b)
            do_group(2 * t + 1, rows_b, gsem_b, wsem_b, rows_a, gsem_a, wsem_a)
            return carry

        lax.fori_loop(0, n_groups // 2, pair_body, 0)
        drain_write(rows_b, wsem_b)

    return gather_kernel


def kernel(x, W):
    b, s = x.shape
    n_total = b * s
    xf = x.reshape(n_total).astype(jnp.int32)
    out = _make_gather(n_total)(W, xf)
    return out.reshape(b, s, D_MODEL)
